# baseline (speedup 1.0000x reference)
import math

import jax
import jax.numpy as jnp
from jax import lax
from jax.experimental import pallas as pl
from jax.experimental.pallas import tpu as pltpu

_CAPACITY_FACTOR = 2.0
_MIN_CAPACITY = 4


def _capacity(s: int, e: int) -> int:
    c = math.floor(_CAPACITY_FACTOR * s / e)
    c += c % 2
    return max(c, _MIN_CAPACITY)


def _cumsum_tokens(x):
    s, e = x.shape
    k = 1
    while k < s:
        shifted = jnp.concatenate(
            [jnp.zeros((k, e), x.dtype), x[: s - k, :]], axis=0)
        x = x + shifted
        k *= 2
    return x


def _router_kernel(x_ref, cb_ref, mask_ref, w1_s, w2_s, r1_s, r2_s):
    pid = pl.program_id(0)
    S, E = x_ref.shape
    T, _, C = cb_ref.shape

    @pl.when(pid == 0)
    def _routing():
        x = x_ref[...]
        m = jnp.max(x, axis=1, keepdims=True)
        u = jnp.exp(x - m)
        p = u / jnp.sum(u, axis=1, keepdims=True)
        ii = lax.broadcasted_iota(jnp.int32, (S, E), 1)
        m1 = jnp.max(p, axis=1, keepdims=True)
        e1 = jnp.min(jnp.where(p == m1, ii, E), axis=1, keepdims=True)
        mask1 = ii == e1
        p2 = jnp.where(mask1, -1.0, p)
        m2 = jnp.max(p2, axis=1, keepdims=True)
        e2 = jnp.min(jnp.where(p2 == m2, ii, E), axis=1, keepdims=True)
        mask2 = ii == e2
        c1 = _cumsum_tokens(mask1.astype(jnp.int32))
        c2 = _cumsum_tokens(mask2.astype(jnp.int32))
        rank1 = c1 - 1
        total1 = jnp.sum(mask1.astype(jnp.int32), axis=0, keepdims=True)
        rank2 = c2 - 1 + total1
        w1_s[...] = jnp.where(mask1 & (rank1 < C), p, 0.0)
        w2_s[...] = jnp.where(mask2 & (rank2 < C), p, 0.0)
        r1_s[...] = rank1
        r2_s[...] = rank2

    t0 = pid * T
    w1 = w1_s[pl.ds(t0, T), :]
    w2 = w2_s[pl.ds(t0, T), :]
    r1 = r1_s[pl.ds(t0, T), :]
    r2 = r2_s[pl.ds(t0, T), :]
    c_iota = lax.broadcasted_iota(jnp.int32, (T, E, C), 2)
    val = (jnp.where(c_iota == r1[:, :, None], w1[:, :, None], 0.0)
           + jnp.where(c_iota == r2[:, :, None], w2[:, :, None], 0.0))
    cb_ref[...] = val
    mask_ref[...] = val != 0.0


def kernel(inputs):
    S, E = inputs.shape
    C = _capacity(S, E)
    T = 256
    grid = (S // T,)
    cb, mask = pl.pallas_call(
        _router_kernel,
        grid=grid,
        in_specs=[pl.BlockSpec((S, E), lambda i: (0, 0))],
        out_specs=[
            pl.BlockSpec((T, E, C), lambda i: (i, 0, 0)),
            pl.BlockSpec((T, E, C), lambda i: (i, 0, 0)),
        ],
        out_shape=[
            jax.ShapeDtypeStruct((S, E, C), jnp.float32),
            jax.ShapeDtypeStruct((S, E, C), jnp.bool_),
        ],
        scratch_shapes=[
            pltpu.VMEM((S, E), jnp.float32),
            pltpu.VMEM((S, E), jnp.float32),
            pltpu.VMEM((S, E), jnp.int32),
            pltpu.VMEM((S, E), jnp.int32),
        ],
    )(inputs.astype(jnp.float32))
    return (cb, mask)


# speedup vs baseline: 1.0173x; 1.0173x over previous
import math

import jax
import jax.numpy as jnp
from jax import lax
from jax.experimental import pallas as pl
from jax.experimental.pallas import tpu as pltpu

_CAPACITY_FACTOR = 2.0
_MIN_CAPACITY = 4


def _capacity(s: int, e: int) -> int:
    c = math.floor(_CAPACITY_FACTOR * s / e)
    c += c % 2
    return max(c, _MIN_CAPACITY)


def _cumsum_tokens(x):
    s, e = x.shape
    k = 1
    while k < s:
        shifted = jnp.concatenate(
            [jnp.zeros((k, e), x.dtype), x[: s - k, :]], axis=0)
        x = x + shifted
        k *= 2
    return x


def _routing_kernel(x_ref, w_ref, r_ref, *, cap):
    S, E = x_ref.shape
    x = x_ref[...]
    m = jnp.max(x, axis=1, keepdims=True)
    u = jnp.exp(x - m)
    p = u / jnp.sum(u, axis=1, keepdims=True)
    ii = lax.broadcasted_iota(jnp.int32, (S, E), 1)
    m1 = jnp.max(p, axis=1, keepdims=True)
    e1 = jnp.min(jnp.where(p == m1, ii, E), axis=1, keepdims=True)
    mask1 = ii == e1
    p2 = jnp.where(mask1, -1.0, p)
    m2 = jnp.max(p2, axis=1, keepdims=True)
    e2 = jnp.min(jnp.where(p2 == m2, ii, E), axis=1, keepdims=True)
    mask2 = ii == e2
    c1 = _cumsum_tokens(mask1.astype(jnp.int32))
    c2 = _cumsum_tokens(mask2.astype(jnp.int32))
    rank1 = c1 - 1
    total1 = jnp.sum(mask1.astype(jnp.int32), axis=0, keepdims=True)
    rank2 = c2 - 1 + total1
    keep1 = mask1 & (rank1 < cap)
    keep2 = mask2 & (rank2 < cap)
    w = jnp.where(keep1, p, 0.0) + jnp.where(keep2, p, 0.0)
    r = jnp.where(keep1, rank1, jnp.where(keep2, rank2, -1))
    r = jnp.where(w != 0.0, r, -1)
    w_ref[...] = w
    r_ref[...] = r


def _build_kernel(w_ref, r_ref, cb_ref, mask_ref):
    T, E, C = cb_ref.shape
    c_iota = lax.broadcasted_iota(jnp.int32, (T, E, C), 2)
    eq = c_iota == r_ref[...][:, :, None]
    cb_ref[...] = jnp.where(eq, w_ref[...][:, :, None], 0.0)
    mask_ref[...] = eq


def kernel(inputs):
    import functools
    S, E = inputs.shape
    C = _capacity(S, E)
    x = inputs.astype(jnp.float32)

    w, r = pl.pallas_call(
        functools.partial(_routing_kernel, cap=C),
        out_shape=[
            jax.ShapeDtypeStruct((S, E), jnp.float32),
            jax.ShapeDtypeStruct((S, E), jnp.int32),
        ],
    )(x)

    T = 256
    cb, mask = pl.pallas_call(
        _build_kernel,
        grid=(S // T,),
        in_specs=[
            pl.BlockSpec((T, E), lambda i: (i, 0)),
            pl.BlockSpec((T, E), lambda i: (i, 0)),
        ],
        out_specs=[
            pl.BlockSpec((T, E, C), lambda i: (i, 0, 0)),
            pl.BlockSpec((T, E, C), lambda i: (i, 0, 0)),
        ],
        out_shape=[
            jax.ShapeDtypeStruct((S, E, C), jnp.float32),
            jax.ShapeDtypeStruct((S, E, C), jnp.bool_),
        ],
    )(w, r)
    return (cb, mask)


# speedup vs baseline: 1.0187x; 1.0014x over previous
import functools
import math

import jax
import jax.numpy as jnp
from jax import lax
from jax.experimental import pallas as pl

_CAPACITY_FACTOR = 2.0
_MIN_CAPACITY = 4


def _capacity(s: int, e: int) -> int:
    c = math.floor(_CAPACITY_FACTOR * s / e)
    c += c % 2
    return max(c, _MIN_CAPACITY)


def _cumsum_tokens(x):
    s, e = x.shape
    k = 1
    while k < s:
        shifted = jnp.concatenate(
            [jnp.zeros((k, e), x.dtype), x[: s - k, :]], axis=0)
        x = x + shifted
        k *= 2
    return x


def _routing_kernel(x_ref, w_ref, r_ref, *, cap):
    S, E = x_ref.shape
    x = x_ref[...]
    m = jnp.max(x, axis=1, keepdims=True)
    u = jnp.exp(x - m)
    p = u / jnp.sum(u, axis=1, keepdims=True)
    ii = lax.broadcasted_iota(jnp.int32, (S, E), 1)
    m1 = jnp.max(p, axis=1, keepdims=True)
    e1 = jnp.min(jnp.where(p == m1, ii, E), axis=1, keepdims=True)
    mask1 = ii == e1
    p2 = jnp.where(mask1, -1.0, p)
    m2 = jnp.max(p2, axis=1, keepdims=True)
    e2 = jnp.min(jnp.where(p2 == m2, ii, E), axis=1, keepdims=True)
    mask2 = ii == e2
    packed = mask1.astype(jnp.int32) + (mask2.astype(jnp.int32) << 16)
    c = _cumsum_tokens(packed)
    c1 = c & 0xFFFF
    c2 = c >> 16
    rank1 = c1 - 1
    total1 = c1[S - 1:S, :]
    rank2 = c2 - 1 + total1
    keep1 = mask1 & (rank1 < cap)
    keep2 = mask2 & (rank2 < cap)
    w = jnp.where(keep1, p, 0.0) + jnp.where(keep2, p, 0.0)
    r = jnp.where(keep1, rank1, jnp.where(keep2, rank2, -1))
    r = jnp.where(w != 0.0, r, -1)
    w_ref[...] = w
    r_ref[...] = r


def _build_kernel(w_ref, r_ref, cb_ref, mask_ref):
    T, E, C = cb_ref.shape
    c_iota = lax.broadcasted_iota(jnp.int32, (T, E, C), 2)
    eq = c_iota == r_ref[...][:, :, None]
    cb_ref[...] = jnp.where(eq, w_ref[...][:, :, None], 0.0)
    mask_ref[...] = eq


def kernel(inputs):
    S, E = inputs.shape
    C = _capacity(S, E)
    x = inputs.astype(jnp.float32)

    w, r = pl.pallas_call(
        functools.partial(_routing_kernel, cap=C),
        out_shape=[
            jax.ShapeDtypeStruct((S, E), jnp.float32),
            jax.ShapeDtypeStruct((S, E), jnp.int32),
        ],
    )(x)

    T = 256
    cb, mask = pl.pallas_call(
        _build_kernel,
        grid=(S // T,),
        in_specs=[
            pl.BlockSpec((T, E), lambda i: (i, 0)),
            pl.BlockSpec((T, E), lambda i: (i, 0)),
        ],
        out_specs=[
            pl.BlockSpec((T, E, C), lambda i: (i, 0, 0)),
            pl.BlockSpec((T, E, C), lambda i: (i, 0, 0)),
        ],
        out_shape=[
            jax.ShapeDtypeStruct((S, E, C), jnp.float32),
            jax.ShapeDtypeStruct((S, E, C), jnp.bool_),
        ],
    )(w, r)
    return (cb, mask)


# speedup vs baseline: 1.0543x; 1.0350x over previous
import math

import jax
import jax.numpy as jnp
from jax import lax
from jax.experimental import pallas as pl
from jax.experimental.pallas import tpu as pltpu

_CAPACITY_FACTOR = 2.0
_MIN_CAPACITY = 4


def _capacity(s: int, e: int) -> int:
    c = math.floor(_CAPACITY_FACTOR * s / e)
    c += c % 2
    return max(c, _MIN_CAPACITY)


def _cumsum_tokens(x):
    s, e = x.shape
    k = 1
    while k < s:
        shifted = jnp.concatenate(
            [jnp.zeros((k, e), x.dtype), x[: s - k, :]], axis=0)
        x = x + shifted
        k *= 2
    return x


def _router_kernel(x_ref, cb_ref, mask_ref, w_s, r_s, *, cap):
    pid = pl.program_id(0)
    S, E = x_ref.shape
    T, _, C = cb_ref.shape

    @pl.when(pid == 0)
    def _routing():
        x = x_ref[...]
        m = jnp.max(x, axis=1, keepdims=True)
        u = jnp.exp(x - m)
        p = u / jnp.sum(u, axis=1, keepdims=True)
        ii = lax.broadcasted_iota(jnp.int32, (S, E), 1)
        m1 = jnp.max(p, axis=1, keepdims=True)
        e1 = jnp.min(jnp.where(p == m1, ii, E), axis=1, keepdims=True)
        mask1 = ii == e1
        p2 = jnp.where(mask1, -1.0, p)
        m2 = jnp.max(p2, axis=1, keepdims=True)
        e2 = jnp.min(jnp.where(p2 == m2, ii, E), axis=1, keepdims=True)
        mask2 = ii == e2
        packed = mask1.astype(jnp.int32) + (mask2.astype(jnp.int32) << 16)
        c = _cumsum_tokens(packed)
        c1 = c & 0xFFFF
        c2 = c >> 16
        rank1 = c1 - 1
        total1 = c1[S - 1:S, :]
        rank2 = c2 - 1 + total1
        keep1 = mask1 & (rank1 < cap)
        keep2 = mask2 & (rank2 < cap)
        w = jnp.where(keep1, p, 0.0) + jnp.where(keep2, p, 0.0)
        r = jnp.where(keep1, rank1, jnp.where(keep2, rank2, -1))
        r = jnp.where(w != 0.0, r, -1)
        w_s[...] = w
        r_s[...] = r

    t0 = pid * T
    w = w_s[pl.ds(t0, T), :]
    r = r_s[pl.ds(t0, T), :]
    c_iota = lax.broadcasted_iota(jnp.int32, (T, E, C), 2)
    eq = c_iota == r[:, :, None]
    cb_ref[...] = jnp.where(eq, w[:, :, None], 0.0)
    mask_ref[...] = eq


def kernel(inputs):
    import functools
    S, E = inputs.shape
    C = _capacity(S, E)
    T = 256
    cb, mask = pl.pallas_call(
        functools.partial(_router_kernel, cap=C),
        grid=(S // T,),
        in_specs=[pl.BlockSpec((S, E), lambda i: (0, 0))],
        out_specs=[
            pl.BlockSpec((T, E, C), lambda i: (i, 0, 0)),
            pl.BlockSpec((T, E, C), lambda i: (i, 0, 0)),
        ],
        out_shape=[
            jax.ShapeDtypeStruct((S, E, C), jnp.float32),
            jax.ShapeDtypeStruct((S, E, C), jnp.bool_),
        ],
        scratch_shapes=[
            pltpu.VMEM((S, E), jnp.float32),
            pltpu.VMEM((S, E), jnp.int32),
        ],
    )(inputs.astype(jnp.float32))
    return (cb, mask)
